# Initial kernel scaffold; baseline (speedup 1.0000x reference)
import jax
import jax.numpy as jnp
from jax.experimental import pallas as pl


def kernel(x, features):
    raise NotImplementedError("write your pallas kernel here")


# speedup vs baseline: 9.4303x; 9.4303x over previous
import functools

import jax
import jax.numpy as jnp
from jax import lax
from jax.experimental import pallas as pl
from jax.experimental.pallas import tpu as pltpu
from jax.experimental.pallas import tpu_sc as plsc

G = 256
K = 32
N = 100000
NPAD = 102400
ROWS = 800
NB = 2048
NBLK = NPAD // NB
BIGI = 2 ** 30
F0 = 94276


def _fps_body(xx_ref, xy_ref, xz_ref, centers_ref, dist_ref):
    flat = (lax.broadcasted_iota(jnp.int32, (ROWS, 128), 0) * 128
            + lax.broadcasted_iota(jnp.int32, (ROWS, 128), 1))
    valid = flat < N
    lane = lax.broadcasted_iota(jnp.int32, (1, 128), 1)
    dist_ref[...] = jnp.full((ROWS, 128), 1e10, jnp.float32)

    def step(i, f):
        r = f // 128
        c = f % 128
        rowx = xx_ref[pl.ds(r, 1), :]
        rowy = xy_ref[pl.ds(r, 1), :]
        rowz = xz_ref[pl.ds(r, 1), :]
        sel = (lane == c).astype(jnp.float32)
        cx = jnp.sum(rowx * sel, axis=1, keepdims=True)
        cy = jnp.sum(rowy * sel, axis=1, keepdims=True)
        cz = jnp.sum(rowz * sel, axis=1, keepdims=True)
        cvec = (jnp.where(lane == 0, cx, 0.0)
                + jnp.where(lane == 1, cy, 0.0)
                + jnp.where(lane == 2, cz, 0.0))
        centers_ref[pl.ds(i, 1), :] = cvec
        dx = xx_ref[...] - cx
        dy = xy_ref[...] - cy
        dz = xz_ref[...] - cz
        d = (dx * dx + dy * dy) + dz * dz
        dist = jnp.minimum(dist_ref[...], d)
        dist_ref[...] = dist
        dmask = jnp.where(valid, dist, -jnp.inf)
        m = jnp.max(dmask)
        return jnp.min(jnp.where(dmask == m, flat, BIGI))

    lax.fori_loop(0, G, step, jnp.int32(F0))


def _fps_call(xx, xy, xz):
    return pl.pallas_call(
        _fps_body,
        out_shape=jax.ShapeDtypeStruct((G, 128), jnp.float32),
        scratch_shapes=[pltpu.VMEM((ROWS, 128), jnp.float32)],
    )(xx, xy, xz)


def _stage2_body(xt_ref, centers_ref, near_ref, cd_ref, ci_ref):
    b = pl.program_id(0)
    x8 = xt_ref[...]
    X = x8[0:1, :]
    Y = x8[1:2, :]
    Z = x8[2:3, :]
    c8 = centers_ref[:, 0:8]
    cx = c8[:, 0:1]
    cy = c8[:, 1:2]
    cz = c8[:, 2:3]
    s = lax.dot_general(c8, x8, (((1,), (0,)), ((), ())),
                        preferred_element_type=jnp.float32)
    cn = (cx * cx + cy * cy) + cz * cz
    xn = (X * X + Y * Y) + Z * Z
    d = (-2.0 * s + cn) + xn
    gidx = b * NB + lax.broadcasted_iota(jnp.int32, (G, NB), 1)
    d = jnp.where(gidx < N, d, jnp.inf)

    m0 = jnp.min(d, axis=0, keepdims=True)
    ridx = lax.broadcasted_iota(jnp.int32, (G, NB), 0)
    am = jnp.min(jnp.where(d == m0, ridx, BIGI), axis=0, keepdims=True)
    near_ref[...] = jnp.broadcast_to(am, (8, NB))

    kiota = lax.broadcasted_iota(jnp.int32, (G, K), 1)

    def ext(j, carry):
        dcur, cda, cia = carry
        m = jnp.min(dcur, axis=1, keepdims=True)
        gi = jnp.min(jnp.where(dcur == m, gidx, BIGI), axis=1, keepdims=True)
        cda = jnp.where(kiota == j, m, cda)
        cia = jnp.where(kiota == j, gi, cia)
        dcur = jnp.where((dcur == m) & (gidx == gi), jnp.inf, dcur)
        return dcur, cda, cia

    cda0 = jnp.zeros((G, K), jnp.float32)
    cia0 = jnp.zeros((G, K), jnp.int32)
    _, cda, cia = lax.fori_loop(0, K, ext, (d, cda0, cia0))
    cd_ref[0] = cda
    ci_ref[0] = cia


def _stage2_call(xt, centers):
    return pl.pallas_call(
        _stage2_body,
        grid=(NBLK,),
        in_specs=[
            pl.BlockSpec((8, NB), lambda b: (0, b)),
            pl.BlockSpec((G, 128), lambda b: (0, 0)),
        ],
        out_specs=[
            pl.BlockSpec((8, NB), lambda b: (0, b)),
            pl.BlockSpec((1, G, K), lambda b: (b, 0, 0)),
            pl.BlockSpec((1, G, K), lambda b: (b, 0, 0)),
        ],
        out_shape=[
            jax.ShapeDtypeStruct((8, NPAD), jnp.int32),
            jax.ShapeDtypeStruct((NBLK, G, K), jnp.float32),
            jax.ShapeDtypeStruct((NBLK, G, K), jnp.int32),
        ],
    )(xt, centers)


def _merge_body(cd_ref, ci_ref, nidx_ref):
    d0 = cd_ref[...]
    ii = ci_ref[...]

    kiota = lax.broadcasted_iota(jnp.int32, (G, K), 1)

    def ext(j, carry):
        dcur, acc = carry
        m = jnp.min(dcur, axis=1, keepdims=True)
        gi = jnp.min(jnp.where(dcur == m, ii, BIGI), axis=1, keepdims=True)
        acc = jnp.where(kiota == j, gi, acc)
        dcur = jnp.where((dcur == m) & (ii == gi), jnp.inf, dcur)
        return dcur, acc

    _, acc = lax.fori_loop(0, K, ext, (d0, jnp.zeros((G, K), jnp.int32)))
    nidx_ref[...] = acc


def _merge_call(cd, ci):
    return pl.pallas_call(
        _merge_body,
        out_shape=jax.ShapeDtypeStruct((G, K), jnp.int32),
    )(cd, ci)


def _gather_call(idx2d, table):
    info = plsc.get_sparse_core_info()
    nw = info.num_cores * info.num_subcores
    b = G * K
    b_per_w = b // nw
    nchunk = b_per_w // 128
    d = table.shape[-1]
    mesh = plsc.VectorSubcoreMesh(core_axis_name="c", subcore_axis_name="s")

    @functools.partial(
        pl.kernel, mesh=mesh,
        out_type=jax.ShapeDtypeStruct((b, d), jnp.float32),
        scratch_types=[
            pltpu.VMEM((nchunk, 128), jnp.int32),
            pltpu.VMEM((nchunk, 128, d), jnp.float32),
            pltpu.SemaphoreType.DMA,
        ],
    )
    def gk(idx_hbm, table_hbm, out_hbm, idx_v, rows_v, sem):
        wid = lax.axis_index("s") * info.num_cores + lax.axis_index("c")
        pltpu.sync_copy(idx_hbm.at[pl.ds(wid * nchunk, nchunk)], idx_v)
        copies = [
            pltpu.async_copy(table_hbm.at[idx_v.at[k]], rows_v.at[k], sem)
            for k in range(nchunk)
        ]
        for k in range(nchunk):
            copies[k].wait()
            pltpu.sync_copy(rows_v.at[k],
                            out_hbm.at[pl.ds(wid * b_per_w + k * 128, 128)])

    return gk(idx2d, table)


def kernel(x, features):
    xf = x[0]
    xt = jnp.zeros((8, NPAD), jnp.float32).at[:3, :N].set(xf.T)
    xx = xt[0].reshape(ROWS, 128)
    xy = xt[1].reshape(ROWS, 128)
    xz = xt[2].reshape(ROWS, 128)

    centers = _fps_call(xx, xy, xz)
    near8, cd, ci = _stage2_call(xt, centers)
    cd2 = jnp.transpose(cd, (1, 0, 2)).reshape(G, NBLK * K)
    ci2 = jnp.transpose(ci, (1, 0, 2)).reshape(G, NBLK * K)
    nidx = _merge_call(cd2, ci2)

    table = features[0]
    idx2d = nidx.reshape(-1, 128)
    rows = _gather_call(idx2d, table)
    neighbors = rows.reshape(1, G, K, features.shape[-1])
    nearest = near8[0:1, :N]
    return (neighbors, nearest)


# speedup vs baseline: 45.0305x; 4.7751x over previous
import functools

import jax
import jax.numpy as jnp
from jax import lax
from jax.experimental import pallas as pl
from jax.experimental.pallas import tpu as pltpu
from jax.experimental.pallas import tpu_sc as plsc

G = 256
K = 32
N = 100000
NPAD = 102400
ROWS = 800
NB = 2048
NBLK = NPAD // NB
BIGI = 2 ** 30
F0 = 94276


def _fps_body(xx_ref, xy_ref, xz_ref, centers_ref, dist_ref):
    flat = (lax.broadcasted_iota(jnp.int32, (ROWS, 128), 0) * 128
            + lax.broadcasted_iota(jnp.int32, (ROWS, 128), 1))
    valid = flat < N
    lane = lax.broadcasted_iota(jnp.int32, (1, 128), 1)
    dist_ref[...] = jnp.full((ROWS, 128), 1e10, jnp.float32)

    def step(i, f):
        r = f // 128
        c = f % 128
        rowx = xx_ref[pl.ds(r, 1), :]
        rowy = xy_ref[pl.ds(r, 1), :]
        rowz = xz_ref[pl.ds(r, 1), :]
        sel = (lane == c).astype(jnp.float32)
        cx = jnp.sum(rowx * sel, axis=1, keepdims=True)
        cy = jnp.sum(rowy * sel, axis=1, keepdims=True)
        cz = jnp.sum(rowz * sel, axis=1, keepdims=True)
        cvec = (jnp.where(lane == 0, cx, 0.0)
                + jnp.where(lane == 1, cy, 0.0)
                + jnp.where(lane == 2, cz, 0.0))
        centers_ref[pl.ds(i, 1), :] = cvec
        dx = xx_ref[...] - cx
        dy = xy_ref[...] - cy
        dz = xz_ref[...] - cz
        d = (dx * dx + dy * dy) + dz * dz
        dist = jnp.minimum(dist_ref[...], d)
        dist_ref[...] = dist
        dmask = jnp.where(valid, dist, -jnp.inf)
        m = jnp.max(dmask)
        return jnp.min(jnp.where(dmask == m, flat, BIGI))

    lax.fori_loop(0, G, step, jnp.int32(F0))


def _fps_call(xx, xy, xz):
    return pl.pallas_call(
        _fps_body,
        out_shape=jax.ShapeDtypeStruct((G, 128), jnp.float32),
        scratch_shapes=[pltpu.VMEM((ROWS, 128), jnp.float32)],
    )(xx, xy, xz)


def _stage2_body(xt_ref, centers_ref, near_ref, dout_ref):
    b = pl.program_id(0)
    x8 = xt_ref[...]
    X = x8[0:1, :]
    Y = x8[1:2, :]
    Z = x8[2:3, :]
    c8 = centers_ref[:, 0:8]
    cx = c8[:, 0:1]
    cy = c8[:, 1:2]
    cz = c8[:, 2:3]
    s = lax.dot_general(c8, x8, (((1,), (0,)), ((), ())),
                        preferred_element_type=jnp.float32)
    cn = (cx * cx + cy * cy) + cz * cz
    xn = (X * X + Y * Y) + Z * Z
    d = (-2.0 * s + cn) + xn
    gidx = b * NB + lax.broadcasted_iota(jnp.int32, (G, NB), 1)
    d = jnp.where(gidx < N, d, jnp.inf)

    m0 = jnp.min(d, axis=0, keepdims=True)
    ridx = lax.broadcasted_iota(jnp.int32, (G, NB), 0)
    am = jnp.min(jnp.where(d == m0, ridx, BIGI), axis=0, keepdims=True)
    near_ref[...] = jnp.broadcast_to(am, (8, NB))

    dout_ref[...] = d


def _stage2_call(xt, centers):
    return pl.pallas_call(
        _stage2_body,
        grid=(NBLK,),
        in_specs=[
            pl.BlockSpec((8, NB), lambda b: (0, b)),
            pl.BlockSpec((G, 128), lambda b: (0, 0)),
        ],
        out_specs=[
            pl.BlockSpec((8, NB), lambda b: (0, b)),
            pl.BlockSpec((G, NB), lambda b: (0, b)),
        ],
        out_shape=[
            jax.ShapeDtypeStruct((8, NPAD), jnp.int32),
            jax.ShapeDtypeStruct((G, NPAD), jnp.float32),
        ],
    )(xt, centers)


SV = 100
CSV = 64


def _topk_call(dmat):
    info = plsc.get_sparse_core_info()
    nw = info.num_cores * info.num_subcores
    rpw = G // nw
    mesh = plsc.VectorSubcoreMesh(core_axis_name="c", subcore_axis_name="s")

    @functools.partial(
        pl.kernel, mesh=mesh,
        out_type=jax.ShapeDtypeStruct((G * K,), jnp.int32),
        scratch_types=[
            pltpu.VMEM((NPAD,), jnp.float32),
            pltpu.VMEM((SV * 16,), jnp.float32),
            pltpu.VMEM((SV * 16,), jnp.float32),
            pltpu.VMEM((rpw * K,), jnp.int32),
            pltpu.SemaphoreType.DMA,
        ],
    )
    def tk(d_hbm, out_hbm, row, fmv, fiv, res, sem):
        wid = lax.axis_index("s") * info.num_cores + lax.axis_index("c")
        iota = lax.iota(jnp.int32, 16)
        iotaf = iota.astype(jnp.float32)
        inf16 = jnp.full((16,), jnp.inf, jnp.float32)

        def _lane_min(v):
            s = v[0]
            for l in range(1, 16):
                s = jnp.minimum(s, v[l])
            return s

        def do_row(r, _):
            gr = wid * rpw + r
            pltpu.async_copy(d_hbm.at[gr], row, sem).wait()

            def fold(u, _):
                fv, fi = inf16, jnp.zeros((16,), jnp.float32)
                base = (u * 1024).astype(jnp.float32)
                for k in range(CSV):
                    dv = row[pl.ds(u * 1024 + 16 * k, 16)]
                    idxv = jnp.full((16,), base + float(16 * k),
                                    jnp.float32) + iotaf
                    c = dv < fv
                    fi = jnp.where(c, idxv, fi)
                    fv = jnp.minimum(fv, dv)
                fmv[pl.ds(u * 16, 16)] = fv
                fiv[pl.ds(u * 16, 16)] = fi
                return 0

            lax.fori_loop(0, SV, fold, 0)

            def ext(j, carry):
                ia, ib = carry
                mv = inf16
                for t in range(SV):
                    mv = jnp.minimum(mv, fmv[pl.ds(16 * t, 16)])
                ms = _lane_min(mv)
                msv = jnp.full((16,), ms, jnp.float32)
                bi = jnp.full((16,), 1e9, jnp.float32)
                for t in range(SV):
                    v = fmv[pl.ds(16 * t, 16)]
                    bi = jnp.minimum(bi, jnp.where(v == msv,
                                                   fiv[pl.ds(16 * t, 16)], 1e9))
                bs = _lane_min(bi)
                bsv = jnp.full((16,), bs, jnp.float32)
                ia = jnp.where(iota == j, bsv, ia)
                ib = jnp.where(iota == (j - 16), bsv, ib)

                ei = bs.astype(jnp.int32)
                us = ei // 1024
                ls = ei - (ei // 16) * 16
                lmask = iota == jnp.full((16,), ls, jnp.int32)
                basef = (us * 1024).astype(jnp.float32)
                rv, ri = inf16, jnp.zeros((16,), jnp.float32)
                for k in range(CSV):
                    dv = row[pl.ds(us * 1024 + 16 * k, 16)]
                    idxv = jnp.full((16,), basef + float(16 * k),
                                    jnp.float32) + iotaf
                    keep = (dv > msv) | ((dv == msv) & (idxv > bsv))
                    vals = jnp.where(keep & lmask, dv, jnp.inf)
                    c = vals < rv
                    ri = jnp.where(c, idxv, ri)
                    rv = jnp.minimum(rv, vals)
                fv = fmv[pl.ds(us * 16, 16)]
                fi = fiv[pl.ds(us * 16, 16)]
                fmv[pl.ds(us * 16, 16)] = jnp.where(lmask, rv, fv)
                fiv[pl.ds(us * 16, 16)] = jnp.where(lmask, ri, fi)
                return ia, ib

            z = jnp.zeros((16,), jnp.float32)
            ia, ib = lax.fori_loop(0, K, ext, (z, z))
            res[pl.ds(r * K, 16)] = ia.astype(jnp.int32)
            res[pl.ds(r * K + 16, 16)] = ib.astype(jnp.int32)
            return 0

        lax.fori_loop(0, rpw, do_row, 0)
        pltpu.sync_copy(res, out_hbm.at[pl.ds(wid * rpw * K, rpw * K)])

    return tk(dmat)


def _gather_call(idx2d, table):
    info = plsc.get_sparse_core_info()
    nw = info.num_cores * info.num_subcores
    b = G * K
    b_per_w = b // nw
    nchunk = b_per_w // 128
    d = table.shape[-1]
    mesh = plsc.VectorSubcoreMesh(core_axis_name="c", subcore_axis_name="s")

    @functools.partial(
        pl.kernel, mesh=mesh,
        out_type=jax.ShapeDtypeStruct((b, d), jnp.float32),
        scratch_types=[
            pltpu.VMEM((nchunk, 128), jnp.int32),
            pltpu.VMEM((nchunk, 128, d), jnp.float32),
            pltpu.SemaphoreType.DMA,
        ],
    )
    def gk(idx_hbm, table_hbm, out_hbm, idx_v, rows_v, sem):
        wid = lax.axis_index("s") * info.num_cores + lax.axis_index("c")
        pltpu.sync_copy(idx_hbm.at[pl.ds(wid * nchunk, nchunk)], idx_v)
        copies = [
            pltpu.async_copy(table_hbm.at[idx_v.at[k]], rows_v.at[k], sem)
            for k in range(nchunk)
        ]
        for k in range(nchunk):
            copies[k].wait()
            pltpu.sync_copy(rows_v.at[k],
                            out_hbm.at[pl.ds(wid * b_per_w + k * 128, 128)])

    return gk(idx2d, table)


def kernel(x, features):
    xf = x[0]
    xt = jnp.zeros((8, NPAD), jnp.float32).at[:3, :N].set(xf.T)
    xx = xt[0].reshape(ROWS, 128)
    xy = xt[1].reshape(ROWS, 128)
    xz = xt[2].reshape(ROWS, 128)

    centers = _fps_call(xx, xy, xz)
    near8, dmat = _stage2_call(xt, centers)
    nidx = _topk_call(dmat).reshape(G, K)

    table = features[0]
    idx2d = nidx.reshape(-1, 128)
    rows = _gather_call(idx2d, table)
    neighbors = rows.reshape(1, G, K, features.shape[-1])
    nearest = near8[0:1, :N]
    return (neighbors, nearest)


# speedup vs baseline: 45.1444x; 1.0025x over previous
import functools

import jax
import jax.numpy as jnp
from jax import lax
from jax.experimental import pallas as pl
from jax.experimental.pallas import tpu as pltpu
from jax.experimental.pallas import tpu_sc as plsc

G = 256
K = 32
N = 100000
NPAD = 102400
ROWS = 800
NB = 2048
NBLK = NPAD // NB
BIGI = 2 ** 30
F0 = 94276


def _fps_body(xx_ref, xy_ref, xz_ref, centers_ref, dist_ref):
    flat = (lax.broadcasted_iota(jnp.int32, (ROWS, 128), 0) * 128
            + lax.broadcasted_iota(jnp.int32, (ROWS, 128), 1))
    valid = flat < N
    lane = lax.broadcasted_iota(jnp.int32, (1, 128), 1)
    dist_ref[...] = jnp.full((ROWS, 128), 1e10, jnp.float32)

    def step(i, f):
        r = f // 128
        c = f % 128
        rowx = xx_ref[pl.ds(r, 1), :]
        rowy = xy_ref[pl.ds(r, 1), :]
        rowz = xz_ref[pl.ds(r, 1), :]
        sel = (lane == c).astype(jnp.float32)
        cx = jnp.sum(rowx * sel, axis=1, keepdims=True)
        cy = jnp.sum(rowy * sel, axis=1, keepdims=True)
        cz = jnp.sum(rowz * sel, axis=1, keepdims=True)
        cvec = (jnp.where(lane == 0, cx, 0.0)
                + jnp.where(lane == 1, cy, 0.0)
                + jnp.where(lane == 2, cz, 0.0))
        centers_ref[pl.ds(i, 1), :] = cvec
        dx = xx_ref[...] - cx
        dy = xy_ref[...] - cy
        dz = xz_ref[...] - cz
        d = (dx * dx + dy * dy) + dz * dz
        dist = jnp.minimum(dist_ref[...], d)
        dist_ref[...] = dist
        dmask = jnp.where(valid, dist, -jnp.inf)
        m = jnp.max(dmask)
        return jnp.min(jnp.where(dmask == m, flat, BIGI))

    lax.fori_loop(0, G, step, jnp.int32(F0))


def _fps_call(xx, xy, xz):
    return pl.pallas_call(
        _fps_body,
        out_shape=jax.ShapeDtypeStruct((G, 128), jnp.float32),
        scratch_shapes=[pltpu.VMEM((ROWS, 128), jnp.float32)],
    )(xx, xy, xz)


GH = G // 2


def _stage2_half0_body(xt_ref, centers_ref, near_ref, m_ref, dout_ref):
    b = pl.program_id(0)
    x8 = xt_ref[...]
    X = x8[0:1, :]
    Y = x8[1:2, :]
    Z = x8[2:3, :]
    c8 = centers_ref[:, 0:8]
    cx = c8[:, 0:1]
    cy = c8[:, 1:2]
    cz = c8[:, 2:3]
    s = lax.dot_general(c8, x8, (((1,), (0,)), ((), ())),
                        preferred_element_type=jnp.float32)
    cn = (cx * cx + cy * cy) + cz * cz
    xn = (X * X + Y * Y) + Z * Z
    d = (-2.0 * s + cn) + xn
    gidx = b * NB + lax.broadcasted_iota(jnp.int32, (GH, NB), 1)
    d = jnp.where(gidx < N, d, jnp.inf)

    m0 = jnp.min(d, axis=0, keepdims=True)
    ridx = lax.broadcasted_iota(jnp.int32, (GH, NB), 0)
    am = jnp.min(jnp.where(d == m0, ridx, BIGI), axis=0, keepdims=True)
    near_ref[...] = jnp.broadcast_to(am, (8, NB))
    m_ref[...] = jnp.broadcast_to(m0, (8, NB))
    dout_ref[...] = d


def _stage2_half1_body(xt_ref, centers_ref, near0_ref, m0_ref,
                       near_ref, dout_ref):
    b = pl.program_id(0)
    x8 = xt_ref[...]
    X = x8[0:1, :]
    Y = x8[1:2, :]
    Z = x8[2:3, :]
    c8 = centers_ref[:, 0:8]
    cx = c8[:, 0:1]
    cy = c8[:, 1:2]
    cz = c8[:, 2:3]
    s = lax.dot_general(c8, x8, (((1,), (0,)), ((), ())),
                        preferred_element_type=jnp.float32)
    cn = (cx * cx + cy * cy) + cz * cz
    xn = (X * X + Y * Y) + Z * Z
    d = (-2.0 * s + cn) + xn
    gidx = b * NB + lax.broadcasted_iota(jnp.int32, (GH, NB), 1)
    d = jnp.where(gidx < N, d, jnp.inf)

    m1 = jnp.min(d, axis=0, keepdims=True)
    ridx = lax.broadcasted_iota(jnp.int32, (GH, NB), 0)
    am1 = jnp.min(jnp.where(d == m1, ridx, BIGI), axis=0, keepdims=True) + GH
    am0 = near0_ref[0:1, :]
    m0 = m0_ref[0:1, :]
    am = jnp.where(m1 < m0, am1, am0)
    near_ref[...] = jnp.broadcast_to(am, (8, NB))
    dout_ref[...] = d


def _stage2_call(xt, centers):
    near0, m0, d0 = pl.pallas_call(
        _stage2_half0_body,
        grid=(NBLK,),
        in_specs=[
            pl.BlockSpec((8, NB), lambda b: (0, b)),
            pl.BlockSpec((GH, 128), lambda b: (0, 0)),
        ],
        out_specs=[
            pl.BlockSpec((8, NB), lambda b: (0, b)),
            pl.BlockSpec((8, NB), lambda b: (0, b)),
            pl.BlockSpec((GH, NB), lambda b: (0, b)),
        ],
        out_shape=[
            jax.ShapeDtypeStruct((8, NPAD), jnp.int32),
            jax.ShapeDtypeStruct((8, NPAD), jnp.float32),
            jax.ShapeDtypeStruct((GH, NPAD), jnp.float32),
        ],
    )(xt, centers[:GH])
    near, d1 = pl.pallas_call(
        _stage2_half1_body,
        grid=(NBLK,),
        in_specs=[
            pl.BlockSpec((8, NB), lambda b: (0, b)),
            pl.BlockSpec((GH, 128), lambda b: (0, 0)),
            pl.BlockSpec((8, NB), lambda b: (0, b)),
            pl.BlockSpec((8, NB), lambda b: (0, b)),
        ],
        out_specs=[
            pl.BlockSpec((8, NB), lambda b: (0, b)),
            pl.BlockSpec((GH, NB), lambda b: (0, b)),
        ],
        out_shape=[
            jax.ShapeDtypeStruct((8, NPAD), jnp.int32),
            jax.ShapeDtypeStruct((GH, NPAD), jnp.float32),
        ],
    )(xt, centers[GH:], near0, m0)
    return near, d0, d1


SV = 100
CSV = 64


def _topk_call(dmat, gr):
    info = plsc.get_sparse_core_info()
    nw = info.num_cores * info.num_subcores
    rpw = gr // nw
    mesh = plsc.VectorSubcoreMesh(core_axis_name="c", subcore_axis_name="s")

    @functools.partial(
        pl.kernel, mesh=mesh,
        out_type=jax.ShapeDtypeStruct((gr * K,), jnp.int32),
        scratch_types=[
            pltpu.VMEM((NPAD,), jnp.float32),
            pltpu.VMEM((SV * 16,), jnp.float32),
            pltpu.VMEM((SV * 16,), jnp.float32),
            pltpu.VMEM((rpw * K,), jnp.int32),
            pltpu.SemaphoreType.DMA,
        ],
    )
    def tk(d_hbm, out_hbm, row, fmv, fiv, res, sem):
        wid = lax.axis_index("s") * info.num_cores + lax.axis_index("c")
        iota = lax.iota(jnp.int32, 16)
        iotaf = iota.astype(jnp.float32)
        inf16 = jnp.full((16,), jnp.inf, jnp.float32)

        def _lane_min(v):
            s = v[0]
            for l in range(1, 16):
                s = jnp.minimum(s, v[l])
            return s

        def do_row(r, _):
            gr = wid * rpw + r
            pltpu.async_copy(d_hbm.at[gr], row, sem).wait()

            def fold(u, _):
                fv, fi = inf16, jnp.zeros((16,), jnp.float32)
                base = (u * 1024).astype(jnp.float32)
                for k in range(CSV):
                    dv = row[pl.ds(u * 1024 + 16 * k, 16)]
                    idxv = jnp.full((16,), base + float(16 * k),
                                    jnp.float32) + iotaf
                    c = dv < fv
                    fi = jnp.where(c, idxv, fi)
                    fv = jnp.minimum(fv, dv)
                fmv[pl.ds(u * 16, 16)] = fv
                fiv[pl.ds(u * 16, 16)] = fi
                return 0

            lax.fori_loop(0, SV, fold, 0)

            def ext(j, carry):
                ia, ib = carry
                mv = inf16
                for t in range(SV):
                    mv = jnp.minimum(mv, fmv[pl.ds(16 * t, 16)])
                ms = _lane_min(mv)
                msv = jnp.full((16,), ms, jnp.float32)
                bi = jnp.full((16,), 1e9, jnp.float32)
                for t in range(SV):
                    v = fmv[pl.ds(16 * t, 16)]
                    bi = jnp.minimum(bi, jnp.where(v == msv,
                                                   fiv[pl.ds(16 * t, 16)], 1e9))
                bs = _lane_min(bi)
                bsv = jnp.full((16,), bs, jnp.float32)
                ia = jnp.where(iota == j, bsv, ia)
                ib = jnp.where(iota == (j - 16), bsv, ib)

                ei = bs.astype(jnp.int32)
                us = ei // 1024
                ls = ei - (ei // 16) * 16
                lmask = iota == jnp.full((16,), ls, jnp.int32)
                basef = (us * 1024).astype(jnp.float32)
                rv, ri = inf16, jnp.zeros((16,), jnp.float32)
                for k in range(CSV):
                    dv = row[pl.ds(us * 1024 + 16 * k, 16)]
                    idxv = jnp.full((16,), basef + float(16 * k),
                                    jnp.float32) + iotaf
                    keep = (dv > msv) | ((dv == msv) & (idxv > bsv))
                    vals = jnp.where(keep & lmask, dv, jnp.inf)
                    c = vals < rv
                    ri = jnp.where(c, idxv, ri)
                    rv = jnp.minimum(rv, vals)
                fv = fmv[pl.ds(us * 16, 16)]
                fi = fiv[pl.ds(us * 16, 16)]
                fmv[pl.ds(us * 16, 16)] = jnp.where(lmask, rv, fv)
                fiv[pl.ds(us * 16, 16)] = jnp.where(lmask, ri, fi)
                return ia, ib

            z = jnp.zeros((16,), jnp.float32)
            ia, ib = lax.fori_loop(0, K, ext, (z, z))
            res[pl.ds(r * K, 16)] = ia.astype(jnp.int32)
            res[pl.ds(r * K + 16, 16)] = ib.astype(jnp.int32)
            return 0

        lax.fori_loop(0, rpw, do_row, 0)
        pltpu.sync_copy(res, out_hbm.at[pl.ds(wid * rpw * K, rpw * K)])

    return tk(dmat)


def _gather_call(idx2d, table):
    info = plsc.get_sparse_core_info()
    nw = info.num_cores * info.num_subcores
    b = G * K
    b_per_w = b // nw
    nchunk = b_per_w // 128
    d = table.shape[-1]
    mesh = plsc.VectorSubcoreMesh(core_axis_name="c", subcore_axis_name="s")

    @functools.partial(
        pl.kernel, mesh=mesh,
        out_type=jax.ShapeDtypeStruct((b, d), jnp.float32),
        scratch_types=[
            pltpu.VMEM((nchunk, 128), jnp.int32),
            pltpu.VMEM((nchunk, 128, d), jnp.float32),
            pltpu.SemaphoreType.DMA,
        ],
    )
    def gk(idx_hbm, table_hbm, out_hbm, idx_v, rows_v, sem):
        wid = lax.axis_index("s") * info.num_cores + lax.axis_index("c")
        pltpu.sync_copy(idx_hbm.at[pl.ds(wid * nchunk, nchunk)], idx_v)
        copies = [
            pltpu.async_copy(table_hbm.at[idx_v.at[k]], rows_v.at[k], sem)
            for k in range(nchunk)
        ]
        for k in range(nchunk):
            copies[k].wait()
            pltpu.sync_copy(rows_v.at[k],
                            out_hbm.at[pl.ds(wid * b_per_w + k * 128, 128)])

    return gk(idx2d, table)


def kernel(x, features):
    xf = x[0]
    xt = jnp.zeros((8, NPAD), jnp.float32).at[:3, :N].set(xf.T)
    xx = xt[0].reshape(ROWS, 128)
    xy = xt[1].reshape(ROWS, 128)
    xz = xt[2].reshape(ROWS, 128)

    centers = _fps_call(xx, xy, xz)
    near8, d0, d1 = _stage2_call(xt, centers)
    n0 = _topk_call(d0, GH)
    n1 = _topk_call(d1, GH)
    nidx = jnp.concatenate([n0, n1]).reshape(G, K)

    table = features[0]
    idx2d = nidx.reshape(-1, 128)
    rows = _gather_call(idx2d, table)
    neighbors = rows.reshape(1, G, K, features.shape[-1])
    nearest = near8[0:1, :N]
    return (neighbors, nearest)


# speedup vs baseline: 47.8696x; 1.0604x over previous
import functools

import jax
import jax.numpy as jnp
from jax import lax
from jax.experimental import pallas as pl
from jax.experimental.pallas import tpu as pltpu
from jax.experimental.pallas import tpu_sc as plsc

G = 256
K = 32
N = 100000
NPAD = 102400
ROWS = 800
NB = 2048
NBLK = NPAD // NB
BIGI = 2 ** 30
F0 = 94276


def _fps_body(xx_ref, xy_ref, xz_ref, centers_ref, dist_ref):
    flat = (lax.broadcasted_iota(jnp.int32, (ROWS, 128), 0) * 128
            + lax.broadcasted_iota(jnp.int32, (ROWS, 128), 1))
    valid = flat < N
    lane = lax.broadcasted_iota(jnp.int32, (1, 128), 1)
    dist_ref[...] = jnp.where(valid, jnp.float32(1e10), -jnp.inf)

    def step(i, f):
        r = f // 128
        c = f % 128
        rowx = xx_ref[pl.ds(r, 1), :]
        rowy = xy_ref[pl.ds(r, 1), :]
        rowz = xz_ref[pl.ds(r, 1), :]
        sel = (lane == c).astype(jnp.float32)
        cx = jnp.sum(rowx * sel, axis=1, keepdims=True)
        cy = jnp.sum(rowy * sel, axis=1, keepdims=True)
        cz = jnp.sum(rowz * sel, axis=1, keepdims=True)
        cvec = (jnp.where(lane == 0, cx, 0.0)
                + jnp.where(lane == 1, cy, 0.0)
                + jnp.where(lane == 2, cz, 0.0))
        centers_ref[pl.ds(i, 1), :] = cvec
        dx = xx_ref[...] - cx
        dy = xy_ref[...] - cy
        dz = xz_ref[...] - cz
        d = (dx * dx + dy * dy) + dz * dz
        dist = jnp.minimum(dist_ref[...], d)
        dist_ref[...] = dist
        m = jnp.max(dist)
        return jnp.min(jnp.where(dist == m, flat, BIGI))

    lax.fori_loop(0, G, step, jnp.int32(F0))


def _fps_call(xx, xy, xz):
    return pl.pallas_call(
        _fps_body,
        out_shape=jax.ShapeDtypeStruct((G, 128), jnp.float32),
        scratch_shapes=[pltpu.VMEM((ROWS, 128), jnp.float32)],
    )(xx, xy, xz)


GH = G // 2


def _stage2_half0_body(xt_ref, centers_ref, near_ref, m_ref, dout_ref):
    b = pl.program_id(0)
    x8 = xt_ref[...]
    X = x8[0:1, :]
    Y = x8[1:2, :]
    Z = x8[2:3, :]
    c8 = centers_ref[:, 0:8]
    cx = c8[:, 0:1]
    cy = c8[:, 1:2]
    cz = c8[:, 2:3]
    s = lax.dot_general(c8, x8, (((1,), (0,)), ((), ())),
                        preferred_element_type=jnp.float32)
    cn = (cx * cx + cy * cy) + cz * cz
    xn = (X * X + Y * Y) + Z * Z
    d = (-2.0 * s + cn) + xn
    gidx = b * NB + lax.broadcasted_iota(jnp.int32, (GH, NB), 1)
    d = jnp.where(gidx < N, d, jnp.inf)

    m0 = jnp.min(d, axis=0, keepdims=True)
    ridx = lax.broadcasted_iota(jnp.int32, (GH, NB), 0)
    am = jnp.min(jnp.where(d == m0, ridx, BIGI), axis=0, keepdims=True)
    near_ref[...] = jnp.broadcast_to(am, (8, NB))
    m_ref[...] = jnp.broadcast_to(m0, (8, NB))
    dout_ref[...] = d


def _stage2_half1_body(xt_ref, centers_ref, near0_ref, m0_ref,
                       near_ref, dout_ref):
    b = pl.program_id(0)
    x8 = xt_ref[...]
    X = x8[0:1, :]
    Y = x8[1:2, :]
    Z = x8[2:3, :]
    c8 = centers_ref[:, 0:8]
    cx = c8[:, 0:1]
    cy = c8[:, 1:2]
    cz = c8[:, 2:3]
    s = lax.dot_general(c8, x8, (((1,), (0,)), ((), ())),
                        preferred_element_type=jnp.float32)
    cn = (cx * cx + cy * cy) + cz * cz
    xn = (X * X + Y * Y) + Z * Z
    d = (-2.0 * s + cn) + xn
    gidx = b * NB + lax.broadcasted_iota(jnp.int32, (GH, NB), 1)
    d = jnp.where(gidx < N, d, jnp.inf)

    m1 = jnp.min(d, axis=0, keepdims=True)
    ridx = lax.broadcasted_iota(jnp.int32, (GH, NB), 0)
    am1 = jnp.min(jnp.where(d == m1, ridx, BIGI), axis=0, keepdims=True) + GH
    am0 = near0_ref[0:1, :]
    m0 = m0_ref[0:1, :]
    am = jnp.where(m1 < m0, am1, am0)
    near_ref[...] = jnp.broadcast_to(am, (8, NB))
    dout_ref[...] = d


def _stage2_call(xt, centers):
    near0, m0, d0 = pl.pallas_call(
        _stage2_half0_body,
        grid=(NBLK,),
        in_specs=[
            pl.BlockSpec((8, NB), lambda b: (0, b)),
            pl.BlockSpec((GH, 128), lambda b: (0, 0)),
        ],
        out_specs=[
            pl.BlockSpec((8, NB), lambda b: (0, b)),
            pl.BlockSpec((8, NB), lambda b: (0, b)),
            pl.BlockSpec((GH, NB), lambda b: (0, b)),
        ],
        out_shape=[
            jax.ShapeDtypeStruct((8, NPAD), jnp.int32),
            jax.ShapeDtypeStruct((8, NPAD), jnp.float32),
            jax.ShapeDtypeStruct((GH, NPAD), jnp.float32),
        ],
    )(xt, centers[:GH])
    near, d1 = pl.pallas_call(
        _stage2_half1_body,
        grid=(NBLK,),
        in_specs=[
            pl.BlockSpec((8, NB), lambda b: (0, b)),
            pl.BlockSpec((GH, 128), lambda b: (0, 0)),
            pl.BlockSpec((8, NB), lambda b: (0, b)),
            pl.BlockSpec((8, NB), lambda b: (0, b)),
        ],
        out_specs=[
            pl.BlockSpec((8, NB), lambda b: (0, b)),
            pl.BlockSpec((GH, NB), lambda b: (0, b)),
        ],
        out_shape=[
            jax.ShapeDtypeStruct((8, NPAD), jnp.int32),
            jax.ShapeDtypeStruct((GH, NPAD), jnp.float32),
        ],
    )(xt, centers[GH:], near0, m0)
    return near, d0, d1


SV = 160
CSV = 40
SV2 = SV // 16


def _topk_call(dmat, gr):
    info = plsc.get_sparse_core_info()
    nw = info.num_cores * info.num_subcores
    rpw = gr // nw
    mesh = plsc.VectorSubcoreMesh(core_axis_name="c", subcore_axis_name="s")

    @functools.partial(
        pl.kernel, mesh=mesh,
        out_type=jax.ShapeDtypeStruct((gr * K,), jnp.int32),
        scratch_types=[
            pltpu.VMEM((NPAD,), jnp.float32),
            pltpu.VMEM((SV * 16,), jnp.float32),
            pltpu.VMEM((SV * 16,), jnp.float32),
            pltpu.VMEM((SV2 * 16,), jnp.float32),
            pltpu.VMEM((SV2 * 16,), jnp.float32),
            pltpu.VMEM((rpw * K,), jnp.int32),
            pltpu.SemaphoreType.DMA,
        ],
    )
    def tk(d_hbm, out_hbm, row, fmv, fiv, l2v, l2i, res, sem):
        wid = lax.axis_index("s") * info.num_cores + lax.axis_index("c")
        iota = lax.iota(jnp.int32, 16)
        iotaf = iota.astype(jnp.float32)
        inf16 = jnp.full((16,), jnp.inf, jnp.float32)
        zero16 = jnp.zeros((16,), jnp.float32)

        def refold_l2(w):
            V, I = inf16, zero16
            for t in range(16):
                v = fmv[pl.ds(w * 256 + 16 * t, 16)]
                i = fiv[pl.ds(w * 256 + 16 * t, 16)]
                c = (v < V) | ((v == V) & (i < I))
                I = jnp.where(c, i, I)
                V = jnp.where(c, v, V)
            l2v[pl.ds(w * 16, 16)] = V
            l2i[pl.ds(w * 16, 16)] = I

        def do_row(r, _):
            gr_row = wid * rpw + r
            pltpu.async_copy(d_hbm.at[gr_row], row, sem).wait()

            def fold(u, _):
                fv, fi = inf16, zero16
                base = (u * 640).astype(jnp.float32)
                for k in range(CSV):
                    dv = row[pl.ds(u * 640 + 16 * k, 16)]
                    idxv = jnp.full((16,), base + float(16 * k),
                                    jnp.float32) + iotaf
                    c = dv < fv
                    fi = jnp.where(c, idxv, fi)
                    fv = jnp.minimum(fv, dv)
                fmv[pl.ds(u * 16, 16)] = fv
                fiv[pl.ds(u * 16, 16)] = fi
                return 0

            lax.fori_loop(0, SV, fold, 0)

            def buildl2(w, _):
                refold_l2(w)
                return 0

            lax.fori_loop(0, SV2, buildl2, 0)

            def ext(j, carry):
                ia, ib = carry
                mv, mi = inf16, zero16
                for w in range(SV2):
                    v = l2v[pl.ds(w * 16, 16)]
                    i = l2i[pl.ds(w * 16, 16)]
                    c = (v < mv) | ((v == mv) & (i < mi))
                    mi = jnp.where(c, i, mi)
                    mv = jnp.where(c, v, mv)
                ms, bs = mv[0], mi[0]
                for l in range(1, 16):
                    vl, il = mv[l], mi[l]
                    c = (vl < ms) | ((vl == ms) & (il < bs))
                    ms = jnp.where(c, vl, ms)
                    bs = jnp.where(c, il, bs)
                msv = jnp.full((16,), ms, jnp.float32)
                bsv = jnp.full((16,), bs, jnp.float32)
                ia = jnp.where(iota == j, bsv, ia)
                ib = jnp.where(iota == (j - 16), bsv, ib)

                ei = bs.astype(jnp.int32)
                us = ei // 640
                ls = ei - (ei // 16) * 16
                lmask = iota == jnp.full((16,), ls, jnp.int32)
                basef = (us * 640).astype(jnp.float32)
                rv, ri = inf16, zero16
                for k in range(CSV):
                    dv = row[pl.ds(us * 640 + 16 * k, 16)]
                    idxv = jnp.full((16,), basef + float(16 * k),
                                    jnp.float32) + iotaf
                    keep = (dv > msv) | ((dv == msv) & (idxv > bsv))
                    vals = jnp.where(keep & lmask, dv, jnp.inf)
                    c = vals < rv
                    ri = jnp.where(c, idxv, ri)
                    rv = jnp.minimum(rv, vals)
                fv = fmv[pl.ds(us * 16, 16)]
                fi = fiv[pl.ds(us * 16, 16)]
                fmv[pl.ds(us * 16, 16)] = jnp.where(lmask, rv, fv)
                fiv[pl.ds(us * 16, 16)] = jnp.where(lmask, ri, fi)
                refold_l2(us // 16)
                return ia, ib

            ia, ib = lax.fori_loop(0, K, ext, (zero16, zero16))
            res[pl.ds(r * K, 16)] = ia.astype(jnp.int32)
            res[pl.ds(r * K + 16, 16)] = ib.astype(jnp.int32)
            return 0

        lax.fori_loop(0, rpw, do_row, 0)
        pltpu.sync_copy(res, out_hbm.at[pl.ds(wid * rpw * K, rpw * K)])

    return tk(dmat)


def _gather_call(idx2d, table):
    info = plsc.get_sparse_core_info()
    nw = info.num_cores * info.num_subcores
    b = G * K
    b_per_w = b // nw
    nchunk = b_per_w // 128
    d = table.shape[-1]
    mesh = plsc.VectorSubcoreMesh(core_axis_name="c", subcore_axis_name="s")

    @functools.partial(
        pl.kernel, mesh=mesh,
        out_type=jax.ShapeDtypeStruct((b, d), jnp.float32),
        scratch_types=[
            pltpu.VMEM((nchunk, 128), jnp.int32),
            pltpu.VMEM((nchunk, 128, d), jnp.float32),
            pltpu.SemaphoreType.DMA,
        ],
    )
    def gk(idx_hbm, table_hbm, out_hbm, idx_v, rows_v, sem):
        wid = lax.axis_index("s") * info.num_cores + lax.axis_index("c")
        pltpu.sync_copy(idx_hbm.at[pl.ds(wid * nchunk, nchunk)], idx_v)
        copies = [
            pltpu.async_copy(table_hbm.at[idx_v.at[k]], rows_v.at[k], sem)
            for k in range(nchunk)
        ]
        for k in range(nchunk):
            copies[k].wait()
            pltpu.sync_copy(rows_v.at[k],
                            out_hbm.at[pl.ds(wid * b_per_w + k * 128, 128)])

    return gk(idx2d, table)


def kernel(x, features):
    xf = x[0]
    xt = jnp.zeros((8, NPAD), jnp.float32).at[:3, :N].set(xf.T)
    xx = xt[0].reshape(ROWS, 128)
    xy = xt[1].reshape(ROWS, 128)
    xz = xt[2].reshape(ROWS, 128)

    centers = _fps_call(xx, xy, xz)
    near8, d0, d1 = _stage2_call(xt, centers)
    n0 = _topk_call(d0, GH)
    n1 = _topk_call(d1, GH)
    nidx = jnp.concatenate([n0, n1]).reshape(G, K)

    table = features[0]
    idx2d = nidx.reshape(-1, 128)
    rows = _gather_call(idx2d, table)
    neighbors = rows.reshape(1, G, K, features.shape[-1])
    nearest = near8[0:1, :N]
    return (neighbors, nearest)
